# baseline (device time: 13948 ns/iter reference)
import jax
import jax.numpy as jnp
from jax import lax
from jax.experimental import pallas as pl
from jax.experimental.pallas import tpu as pltpu

N_DEV = 4
N_CHUNK = 4
EPS = 1e-5


def kernel(x, t_emb, W_scale, W_shift):
    b, s, c_per = x.shape
    c_total = c_per * N_DEV
    s_chunk = s // N_CHUNK

    def body(x_ref, t_ref, ws_ref, wsh_ref, out_ref,
             xv_ref, ov_ref, stats_ref,
             in_sems, out_sems, send_sems, recv_sems):
        my_pos = lax.axis_index("i")
        peers = [(my_pos + d) % N_DEV for d in (1, 2, 3)]

        copies_in = []
        for j in range(N_CHUNK):
            cp = pltpu.make_async_copy(
                x_ref.at[:, pl.ds(j * s_chunk, s_chunk), :],
                xv_ref.at[j],
                in_sems.at[j],
            )
            cp.start()
            copies_in.append(cp)

        barrier = pltpu.get_barrier_semaphore()
        for nbr in peers:
            pl.semaphore_signal(
                barrier, inc=1,
                device_id=(nbr,), device_id_type=pl.DeviceIdType.MESH,
            )

        scale = jnp.dot(t_ref[...], ws_ref[...],
                        preferred_element_type=jnp.float32)
        shift = jnp.dot(t_ref[...], wsh_ref[...],
                        preferred_element_type=jnp.float32)
        s1 = (1.0 + scale).astype(jnp.bfloat16)
        sh = shift.astype(jnp.bfloat16)

        ones = jnp.ones((c_per,), jnp.bfloat16)
        dims = (((2,), (0,)), ((), ()))

        xbs, rdmas = [], []
        for j in range(N_CHUNK):
            copies_in[j].wait()
            xb = xv_ref[j].astype(jnp.bfloat16)
            xbs.append(xb)
            stats_ref[j, 0, 0] = lax.dot_general(
                xb, ones, dims, preferred_element_type=jnp.float32)
            stats_ref[j, 0, 1] = lax.dot_general(
                xb * xb, ones, dims, preferred_element_type=jnp.float32)
            if j == 0:
                pl.semaphore_wait(barrier, N_DEV - 1)
            chunk_rdmas = []
            for i, d in enumerate((1, 2, 3)):
                slot = N_DEV - d
                rdma = pltpu.make_async_remote_copy(
                    src_ref=stats_ref.at[j, 0],
                    dst_ref=stats_ref.at[j, slot],
                    send_sem=send_sems.at[j, i],
                    recv_sem=recv_sems.at[j, slot],
                    device_id=(peers[i],),
                    device_id_type=pl.DeviceIdType.MESH,
                )
                rdma.start()
                chunk_rdmas.append(rdma)
            rdmas.append(chunk_rdmas)

        copies_out = []
        for j in range(N_CHUNK):
            for rdma in rdmas[j]:
                rdma.wait_recv()
            tot = (stats_ref[j, 0] + stats_ref[j, 1]
                   + stats_ref[j, 2] + stats_ref[j, 3])
            mean = tot[0] * (1.0 / c_total)
            var = tot[1] * (1.0 / c_total) - mean * mean
            rstd = lax.rsqrt(var + EPS)
            mean_b = mean.astype(jnp.bfloat16)
            rstd_b = rstd.astype(jnp.bfloat16)
            h_norm = (xbs[j] - mean_b[:, :, None]) * rstd_b[:, :, None]
            ov_ref[j] = h_norm * s1[:, None, :] + sh[:, None, :]
            cp = pltpu.make_async_copy(
                ov_ref.at[j],
                out_ref.at[:, pl.ds(j * s_chunk, s_chunk), :],
                out_sems.at[j],
            )
            cp.start()
            copies_out.append(cp)

        for cp in copies_out:
            cp.wait()
        for chunk_rdmas in rdmas:
            for rdma in chunk_rdmas:
                rdma.wait_send()

    return pl.pallas_call(
        body,
        out_shape=jax.ShapeDtypeStruct((b, s, c_per), jnp.bfloat16),
        in_specs=[
            pl.BlockSpec(memory_space=pltpu.MemorySpace.HBM),
            pl.BlockSpec(memory_space=pltpu.VMEM),
            pl.BlockSpec(memory_space=pltpu.VMEM),
            pl.BlockSpec(memory_space=pltpu.VMEM),
        ],
        out_specs=pl.BlockSpec(memory_space=pltpu.MemorySpace.HBM),
        scratch_shapes=[
            pltpu.VMEM((N_CHUNK, b, s_chunk, c_per), jnp.float32),
            pltpu.VMEM((N_CHUNK, b, s_chunk, c_per), jnp.bfloat16),
            pltpu.VMEM((N_CHUNK, N_DEV, 2, b, s_chunk), jnp.float32),
            pltpu.SemaphoreType.DMA((N_CHUNK,)),
            pltpu.SemaphoreType.DMA((N_CHUNK,)),
            pltpu.SemaphoreType.DMA((N_CHUNK, N_DEV - 1)),
            pltpu.SemaphoreType.DMA((N_CHUNK, N_DEV)),
        ],
        compiler_params=pltpu.CompilerParams(collective_id=0),
    )(x, t_emb, W_scale, W_shift)
